# baseline (device time: 34225 ns/iter reference)
import jax
import jax.numpy as jnp
from jax import lax
from jax.experimental import pallas as pl
from jax.experimental.pallas import tpu as pltpu

N_DEV = 16
B, SQ, SKV, HQ, DH = 2, 128, 128, 64, 64
H_PER = HQ // N_DEV
DM = 512
DHEADS = H_PER * DH
N_CHUNK = N_DEV
CPB = N_CHUNK // B
ROWS = SQ // CPB
BF16 = jnp.bfloat16
F32 = jnp.float32


def kernel(x, Wq, K_ext, V_ext, Wo):
    def body(x_ref, wq_ref, k_ref, v_ref, wo_ref, out_ref,
             acc_ref, rs_ref, red_ref, wq_vmem, wo_vmem, local_sems,
             send_a, recv_a, send_b, recv_b):
        me = lax.axis_index("i")

        cp_wq = pltpu.make_async_copy(
            wq_ref.at[:, pl.ds(me * DHEADS, DHEADS)], wq_vmem, local_sems.at[0])
        cp_wo = pltpu.make_async_copy(
            wo_ref.at[pl.ds(me * DHEADS, DHEADS), :], wo_vmem, local_sems.at[1])
        cp_wq.start()
        cp_wo.start()
        cp_wq.wait()
        cp_wo.wait()

        wq_my = wq_vmem[...].astype(BF16)
        for b in range(B):
            xb = x_ref[b].astype(BF16)
            qb = lax.dot_general(xb, wq_my, (((1,), (0,)), ((), ())),
                                 preferred_element_type=F32) * 0.125
            y = jnp.zeros((SQ, DM), F32)
            for h in range(H_PER):
                q = qb[:, h * DH:(h + 1) * DH].astype(BF16)
                k = k_ref[b][:, h, :].astype(BF16)
                v = v_ref[b][:, h, :].astype(BF16)
                s = lax.dot_general(q, k, (((1,), (1,)), ((), ())),
                                    preferred_element_type=F32)
                s = s - jnp.max(s, axis=-1, keepdims=True)
                w = jnp.exp(s)
                w = w / jnp.sum(w, axis=-1, keepdims=True)
                ctx = lax.dot_general(w.astype(BF16), v,
                                      (((1,), (0,)), ((), ())),
                                      preferred_element_type=F32)
                wo_h = wo_vmem[pl.ds(h * DH, DH), :].astype(BF16)
                y = y + lax.dot_general(ctx.astype(BF16), wo_h,
                                        (((1,), (0,)), ((), ())),
                                        preferred_element_type=F32)
            acc_ref[pl.ds(b * CPB, CPB)] = y.astype(BF16).reshape(CPB, ROWS, DM)

        for d in range(1, N_DEV):
            j = (me + d) % N_DEV
            pltpu.make_async_remote_copy(
                src_ref=acc_ref.at[pl.ds(j, 1)],
                dst_ref=rs_ref.at[pl.ds(d, 1)],
                send_sem=send_a.at[d],
                recv_sem=recv_a.at[d],
                device_id=j,
                device_id_type=pl.DeviceIdType.LOGICAL,
            ).start()
        for d in range(1, N_DEV):
            pltpu.make_async_remote_copy(
                src_ref=acc_ref.at[pl.ds(0, 1)],
                dst_ref=rs_ref.at[pl.ds(d, 1)],
                send_sem=send_a.at[d],
                recv_sem=recv_a.at[d],
                device_id=me,
                device_id_type=pl.DeviceIdType.LOGICAL,
            ).wait_recv()

        mine = acc_ref[pl.ds(me, 1)].astype(F32)
        others = rs_ref[pl.ds(1, N_DEV - 1)].astype(F32)
        red = (mine + jnp.sum(others, axis=0, keepdims=True)).astype(BF16)
        red_ref[...] = red

        bm = me // CPB
        rm = (me % CPB) * ROWS
        out_ref[pl.ds(bm, 1), pl.ds(rm, ROWS), :] = red

        for d in range(1, N_DEV):
            j = (me + d) % N_DEV
            pltpu.make_async_remote_copy(
                src_ref=red_ref,
                dst_ref=out_ref.at[pl.ds(bm, 1), pl.ds(rm, ROWS), :],
                send_sem=send_b.at[d],
                recv_sem=recv_b.at[d],
                device_id=j,
                device_id_type=pl.DeviceIdType.LOGICAL,
            ).start()
        for d in range(1, N_DEV):
            s = (me + N_DEV - d) % N_DEV
            bs = s // CPB
            rg = (s % CPB) * ROWS
            pltpu.make_async_remote_copy(
                src_ref=red_ref,
                dst_ref=out_ref.at[pl.ds(bs, 1), pl.ds(rg, ROWS), :],
                send_sem=send_b.at[d],
                recv_sem=recv_b.at[d],
                device_id=me,
                device_id_type=pl.DeviceIdType.LOGICAL,
            ).wait_recv()

        for d in range(1, N_DEV):
            pltpu.make_async_remote_copy(
                src_ref=acc_ref.at[pl.ds(0, 1)],
                dst_ref=rs_ref.at[pl.ds(d, 1)],
                send_sem=send_a.at[d],
                recv_sem=recv_a.at[d],
                device_id=me,
                device_id_type=pl.DeviceIdType.LOGICAL,
            ).wait_send()
            pltpu.make_async_remote_copy(
                src_ref=red_ref,
                dst_ref=out_ref.at[pl.ds(0, 1), pl.ds(0, ROWS), :],
                send_sem=send_b.at[d],
                recv_sem=recv_b.at[d],
                device_id=me,
                device_id_type=pl.DeviceIdType.LOGICAL,
            ).wait_send()

    return pl.pallas_call(
        body,
        out_shape=jax.ShapeDtypeStruct((B, SQ, DM), BF16),
        in_specs=[
            pl.BlockSpec(memory_space=pltpu.VMEM),
            pl.BlockSpec(memory_space=pl.ANY),
            pl.BlockSpec(memory_space=pltpu.VMEM),
            pl.BlockSpec(memory_space=pltpu.VMEM),
            pl.BlockSpec(memory_space=pl.ANY),
        ],
        out_specs=pl.BlockSpec(memory_space=pltpu.VMEM),
        scratch_shapes=[
            pltpu.VMEM((N_CHUNK, ROWS, DM), BF16),
            pltpu.VMEM((N_DEV, ROWS, DM), BF16),
            pltpu.VMEM((1, ROWS, DM), BF16),
            pltpu.VMEM((DM, DHEADS), F32),
            pltpu.VMEM((DHEADS, DM), F32),
            pltpu.SemaphoreType.DMA((2,)),
            pltpu.SemaphoreType.DMA((N_DEV,)),
            pltpu.SemaphoreType.DMA((N_DEV,)),
            pltpu.SemaphoreType.DMA((N_DEV,)),
            pltpu.SemaphoreType.DMA((N_DEV,)),
        ],
    )(x, Wq, K_ext, V_ext, Wo)


# device time: 21558 ns/iter; 1.5876x vs baseline; 1.5876x over previous
import jax
import jax.numpy as jnp
from jax import lax
from jax.experimental import pallas as pl
from jax.experimental.pallas import tpu as pltpu

N_DEV = 16
B, SQ, SKV, HQ, DH = 2, 128, 128, 64, 64
H_PER = HQ // N_DEV
DM = 512
DHEADS = H_PER * DH
N_CHUNK = N_DEV
CPB = N_CHUNK // B
ROWS = SQ // CPB
BF16 = jnp.bfloat16
F32 = jnp.float32


def kernel(x, Wq, K_ext, V_ext, Wo):
    me_out = lax.axis_index("i")
    wq_my = lax.dynamic_slice(Wq, (0, me_out * DHEADS), (DM, DHEADS))
    wo_my = lax.dynamic_slice(Wo, (me_out * DHEADS, 0), (DHEADS, DM))
    wq16 = wq_my.astype(BF16)
    wo16 = wo_my.astype(BF16)
    k16 = K_ext.reshape(B, SKV, DHEADS).astype(BF16)
    v16 = V_ext.reshape(B, SKV, DHEADS).astype(BF16)

    def body(x_ref, wq_ref, k_ref, v_ref, wo_ref, out_ref,
             acc_ref, rs_ref, send_a, recv_a, send_b, recv_b):
        me = lax.axis_index("i")

        barrier = pltpu.get_barrier_semaphore()
        for d in range(1, N_DEV):
            pl.semaphore_signal(
                barrier, inc=1, device_id=(me + d) % N_DEV,
                device_id_type=pl.DeviceIdType.LOGICAL)

        wq = wq_ref[...]

        def compute_batch(b):
            xb = x_ref[b].astype(BF16)
            qb = lax.dot_general(xb, wq, (((1,), (0,)), ((), ())),
                                 preferred_element_type=F32) * 0.125
            kb = k_ref[b]
            vb = v_ref[b]
            y = jnp.zeros((SQ, DM), F32)
            for h in range(H_PER):
                q = qb[:, h * DH:(h + 1) * DH].astype(BF16)
                k = kb[:, h * DH:(h + 1) * DH]
                v = vb[:, h * DH:(h + 1) * DH]
                s = lax.dot_general(q, k, (((1,), (1,)), ((), ())),
                                    preferred_element_type=F32)
                w = jnp.exp(s)
                denom = jnp.sum(w, axis=-1, keepdims=True)
                ctx = lax.dot_general(w.astype(BF16), v,
                                      (((1,), (0,)), ((), ())),
                                      preferred_element_type=F32)
                ctx = ctx / denom
                wo_h = wo_ref[h * DH:(h + 1) * DH, :]
                y = y + lax.dot_general(ctx.astype(BF16), wo_h,
                                        (((1,), (0,)), ((), ())),
                                        preferred_element_type=F32)
            acc_ref[pl.ds(b * CPB, CPB)] = y.astype(BF16).reshape(CPB, ROWS, DM)

        compute_batch(0)
        compute_batch(1)
        pl.semaphore_wait(barrier, N_DEV - 1)

        for d in range(1, N_DEV):
            j = (me + d) % N_DEV
            pltpu.make_async_remote_copy(
                src_ref=acc_ref.at[pl.ds(j, 1)],
                dst_ref=rs_ref.at[pl.ds(d, 1)],
                send_sem=send_a.at[d],
                recv_sem=recv_a.at[d],
                device_id=j,
                device_id_type=pl.DeviceIdType.LOGICAL,
            ).start()
        for d in range(1, N_DEV):
            pltpu.make_async_remote_copy(
                src_ref=acc_ref.at[pl.ds(0, 1)],
                dst_ref=rs_ref.at[pl.ds(d, 1)],
                send_sem=send_a.at[d],
                recv_sem=recv_a.at[d],
                device_id=me,
                device_id_type=pl.DeviceIdType.LOGICAL,
            ).wait_recv()

        mine = acc_ref[pl.ds(me, 1)].astype(F32)
        others = rs_ref[pl.ds(1, N_DEV - 1)].astype(F32)
        red = (mine + jnp.sum(others, axis=0, keepdims=True)).astype(BF16)

        bm = me // CPB
        rm = (me % CPB) * ROWS
        out_ref[pl.ds(bm, 1), pl.ds(rm, ROWS), :] = red

        for d in range(1, N_DEV):
            j = (me + d) % N_DEV
            pltpu.make_async_remote_copy(
                src_ref=out_ref.at[pl.ds(bm, 1), pl.ds(rm, ROWS), :],
                dst_ref=out_ref.at[pl.ds(bm, 1), pl.ds(rm, ROWS), :],
                send_sem=send_b.at[d],
                recv_sem=recv_b.at[d],
                device_id=j,
                device_id_type=pl.DeviceIdType.LOGICAL,
            ).start()
        for d in range(1, N_DEV):
            s = (me + N_DEV - d) % N_DEV
            bs = s // CPB
            rg = (s % CPB) * ROWS
            pltpu.make_async_remote_copy(
                src_ref=out_ref.at[pl.ds(bm, 1), pl.ds(rm, ROWS), :],
                dst_ref=out_ref.at[pl.ds(bs, 1), pl.ds(rg, ROWS), :],
                send_sem=send_b.at[d],
                recv_sem=recv_b.at[d],
                device_id=me,
                device_id_type=pl.DeviceIdType.LOGICAL,
            ).wait_recv()

        for d in range(1, N_DEV):
            pltpu.make_async_remote_copy(
                src_ref=acc_ref.at[pl.ds(0, 1)],
                dst_ref=rs_ref.at[pl.ds(d, 1)],
                send_sem=send_a.at[d],
                recv_sem=recv_a.at[d],
                device_id=me,
                device_id_type=pl.DeviceIdType.LOGICAL,
            ).wait_send()
            pltpu.make_async_remote_copy(
                src_ref=out_ref.at[pl.ds(bm, 1), pl.ds(rm, ROWS), :],
                dst_ref=out_ref.at[pl.ds(0, 1), pl.ds(0, ROWS), :],
                send_sem=send_b.at[d],
                recv_sem=recv_b.at[d],
                device_id=me,
                device_id_type=pl.DeviceIdType.LOGICAL,
            ).wait_send()

    return pl.pallas_call(
        body,
        out_shape=jax.ShapeDtypeStruct((B, SQ, DM), BF16),
        in_specs=[pl.BlockSpec(memory_space=pltpu.VMEM)] * 5,
        out_specs=pl.BlockSpec(memory_space=pltpu.VMEM),
        scratch_shapes=[
            pltpu.VMEM((N_CHUNK, ROWS, DM), BF16),
            pltpu.VMEM((N_DEV, ROWS, DM), BF16),
            pltpu.SemaphoreType.DMA((N_DEV,)),
            pltpu.SemaphoreType.DMA((N_DEV,)),
            pltpu.SemaphoreType.DMA((N_DEV,)),
            pltpu.SemaphoreType.DMA((N_DEV,)),
        ],
        compiler_params=pltpu.CompilerParams(collective_id=0),
    )(x, wq16, k16, v16, wo16)
